# baseline (speedup 1.0000x reference)
import functools

import jax
import jax.numpy as jnp
from jax import lax
from jax.experimental import pallas as pl
from jax.experimental.pallas import tpu as pltpu
from jax.experimental.pallas import tpu_sc as plsc

N = 100000
D = 128
LANES = 16
NCORES = 2
NSUB = 16
NW = NCORES * NSUB
CHUNK = 160
NCHUNK = N // CHUNK
GROUPS = CHUNK // LANES
RING = 4


def _sc_body(p_hbm, tab_hbm, out_hbm, p_v, o_v, t_v, psem, osem):
    w = lax.axis_index("s") * NCORES + lax.axis_index("c")
    nc = (NCHUNK - w + NW - 1) // NW

    pltpu.sync_copy(tab_hbm.at[pl.ds(0, 2)], t_v)
    t0s = [t_v[0, pl.ds(16 * j, 16)] for j in range(D // LANES)]
    dls = [t_v[1, pl.ds(16 * j, 16)] - t0s[j] for j in range(D // LANES)]

    def base(k):
        return (w + k * NW) * CHUNK

    pltpu.async_copy(p_hbm.at[pl.ds(base(0), CHUNK)], p_v.at[pl.ds(0, CHUNK)],
                     psem)

    def chunk_body(k, carry):
        off = lax.rem(k, RING) * CHUNK
        pb = p_v.at[pl.ds(off, CHUNK)]
        ob = o_v.at[pl.ds(off, CHUNK)]

        pltpu.make_async_copy(
            p_hbm.at[pl.ds(base(k), CHUNK)], pb, psem).wait()

        @pl.when(k + 1 < nc)
        def _():
            noff = lax.rem(k + 1, RING) * CHUNK
            pltpu.async_copy(
                p_hbm.at[pl.ds(base(k + 1), CHUNK)],
                p_v.at[pl.ds(noff, CHUNK)], psem)

        @pl.when(k >= RING)
        def _():
            pltpu.make_async_copy(
                ob, out_hbm.at[pl.ds(base(k), CHUNK)], osem).wait()

        def group_body(g, c):
            pv = pb[pl.ds(LANES * g, LANES)]
            for r in range(LANES):
                pi = pv[r]
                i = LANES * g + r
                for j in range(D // LANES):
                    ob[i, pl.ds(16 * j, 16)] = t0s[j] + pi * dls[j]
            return c

        lax.fori_loop(0, GROUPS, group_body, 0)
        pltpu.async_copy(ob, out_hbm.at[pl.ds(base(k), CHUNK)], osem)
        return carry

    lax.fori_loop(0, nc, chunk_body, 0)

    ob0 = o_v.at[pl.ds(0, CHUNK)]
    for _ in range(RING):
        pltpu.make_async_copy(ob0, out_hbm.at[pl.ds(0, CHUNK)], osem).wait()


def kernel(label_probs, table):
    mesh = plsc.VectorSubcoreMesh(core_axis_name="c", subcore_axis_name="s")
    f = functools.partial(
        pl.kernel,
        out_type=jax.ShapeDtypeStruct((N, D), jnp.float32),
        mesh=mesh,
        scratch_types=[
            pltpu.VMEM((RING * CHUNK,), jnp.float32),
            pltpu.VMEM((RING * CHUNK, D), jnp.float32),
            pltpu.VMEM((2, D), jnp.float32),
            pltpu.SemaphoreType.DMA,
            pltpu.SemaphoreType.DMA,
        ],
    )(_sc_body)
    return f(label_probs, table)


# speedup vs baseline: 1.0072x; 1.0072x over previous
import functools

import jax
import jax.numpy as jnp
from jax import lax
from jax.experimental import pallas as pl
from jax.experimental.pallas import tpu as pltpu
from jax.experimental.pallas import tpu_sc as plsc

N = 100000
D = 128
LANES = 16
NCORES = 2
NSUB = 16
NW = NCORES * NSUB
CHUNK = 400
NCHUNK = N // CHUNK
GROUPS = CHUNK // LANES
MAXC = NCHUNK // NW + 1
RING = 2


def _sc_body(p_hbm, tab_hbm, out_hbm, p_v, o_v, t_v, osem):
    w = lax.axis_index("s") * NCORES + lax.axis_index("c")
    nc = (NCHUNK - w + NW - 1) // NW
    s0 = jnp.where(w < 26, 8 * w, 7 * w + 26)
    start = jnp.minimum(s0, NCHUNK - MAXC)
    delta = s0 - start

    pltpu.sync_copy(tab_hbm.at[pl.ds(0, 2)], t_v)
    pltpu.sync_copy(p_hbm.at[pl.ds(start * CHUNK, MAXC * CHUNK)], p_v)

    t0s = [t_v[0, pl.ds(16 * j, 16)] for j in range(D // LANES)]
    dls = [t_v[1, pl.ds(16 * j, 16)] - t0s[j] for j in range(D // LANES)]

    def chunk_body(k, carry):
        off = lax.rem(k, RING) * CHUNK
        pb = p_v.at[pl.ds((k + delta) * CHUNK, CHUNK)]
        ob = o_v.at[pl.ds(off, CHUNK)]

        @pl.when(k >= RING)
        def _():
            pltpu.make_async_copy(
                ob, out_hbm.at[pl.ds(0, CHUNK)], osem).wait()

        def group_body(g, c):
            pv = pb[pl.ds(LANES * g, LANES)]
            for r in range(LANES):
                pi = pv[r]
                i = LANES * g + r
                for j in range(D // LANES):
                    ob[i, pl.ds(16 * j, 16)] = t0s[j] + pi * dls[j]
            return c

        lax.fori_loop(0, GROUPS, group_body, 0)
        pltpu.async_copy(ob, out_hbm.at[pl.ds((s0 + k) * CHUNK, CHUNK)], osem)
        return carry

    lax.fori_loop(0, nc, chunk_body, 0)

    ob0 = o_v.at[pl.ds(0, CHUNK)]
    for _ in range(RING):
        pltpu.make_async_copy(ob0, out_hbm.at[pl.ds(0, CHUNK)], osem).wait()


def kernel(label_probs, table):
    mesh = plsc.VectorSubcoreMesh(core_axis_name="c", subcore_axis_name="s")
    f = functools.partial(
        pl.kernel,
        out_type=jax.ShapeDtypeStruct((N, D), jnp.float32),
        mesh=mesh,
        scratch_types=[
            pltpu.VMEM((MAXC * CHUNK,), jnp.float32),
            pltpu.VMEM((RING * CHUNK, D), jnp.float32),
            pltpu.VMEM((2, D), jnp.float32),
            pltpu.SemaphoreType.DMA,
        ],
    )(_sc_body)
    return f(label_probs, table)


# speedup vs baseline: 1.0224x; 1.0151x over previous
import functools

import jax
import jax.numpy as jnp
from jax import lax
from jax.experimental import pallas as pl
from jax.experimental.pallas import tpu as pltpu
from jax.experimental.pallas import tpu_sc as plsc

N = 100000
D = 128
LANES = 16
NCORES = 2
NSUB = 16
NW = NCORES * NSUB
CHUNK = 400
NCHUNK = N // CHUNK
GROUPS = CHUNK // LANES
MAXC = NCHUNK // NW + 1
RING = 2


def _sc_body(p_hbm, tab_hbm, out_hbm, p_v, o_v, t_v, osem):
    w = lax.axis_index("s") * NCORES + lax.axis_index("c")
    nc = (NCHUNK - w + NW - 1) // NW
    s0 = jnp.where(w < 26, 8 * w, 7 * w + 26)
    start = jnp.minimum(s0, NCHUNK - MAXC)
    delta = s0 - start

    pltpu.sync_copy(tab_hbm.at[pl.ds(0, 2)], t_v)
    pltpu.sync_copy(p_hbm.at[pl.ds(start * CHUNK, MAXC * CHUNK)], p_v)

    t0s = [t_v[0, pl.ds(16 * j, 16)] for j in range(D // LANES)]
    dls = [t_v[1, pl.ds(16 * j, 16)] - t0s[j] for j in range(D // LANES)]

    H0 = 208
    H1 = CHUNK - H0

    def chunk_body(k, carry):
        off = lax.rem(k, RING) * CHUNK
        pb = p_v.at[pl.ds((k + delta) * CHUNK, CHUNK)]
        ob = o_v.at[pl.ds(off, CHUNK)]

        def group_range(g0, g1):
            def group_body(g, c):
                pv = pb[pl.ds(LANES * g, LANES)]
                for r in range(LANES):
                    pi = pv[r]
                    i = LANES * g + r
                    for j in range(D // LANES):
                        ob[i, pl.ds(16 * j, 16)] = t0s[j] + pi * dls[j]
                return c
            lax.fori_loop(g0, g1, group_body, 0)

        @pl.when(k >= RING)
        def _():
            pltpu.make_async_copy(
                ob.at[pl.ds(0, H0)], out_hbm.at[pl.ds(0, H0)], osem).wait()

        group_range(0, H0 // LANES)
        pltpu.async_copy(
            ob.at[pl.ds(0, H0)],
            out_hbm.at[pl.ds((s0 + k) * CHUNK, H0)], osem)

        @pl.when(k >= RING)
        def _():
            pltpu.make_async_copy(
                ob.at[pl.ds(H0, H1)], out_hbm.at[pl.ds(0, H1)], osem).wait()

        group_range(H0 // LANES, GROUPS)
        pltpu.async_copy(
            ob.at[pl.ds(H0, H1)],
            out_hbm.at[pl.ds((s0 + k) * CHUNK + H0, H1)], osem)
        return carry

    lax.fori_loop(0, nc, chunk_body, 0)

    ob0 = o_v.at[pl.ds(0, CHUNK)]
    for _ in range(RING):
        pltpu.make_async_copy(
            ob0.at[pl.ds(0, H0)], out_hbm.at[pl.ds(0, H0)], osem).wait()
        pltpu.make_async_copy(
            ob0.at[pl.ds(H0, H1)], out_hbm.at[pl.ds(0, H1)], osem).wait()


def kernel(label_probs, table):
    mesh = plsc.VectorSubcoreMesh(core_axis_name="c", subcore_axis_name="s")
    f = functools.partial(
        pl.kernel,
        out_type=jax.ShapeDtypeStruct((N, D), jnp.float32),
        mesh=mesh,
        scratch_types=[
            pltpu.VMEM((MAXC * CHUNK,), jnp.float32),
            pltpu.VMEM((RING * CHUNK, D), jnp.float32),
            pltpu.VMEM((2, D), jnp.float32),
            pltpu.SemaphoreType.DMA,
        ],
    )(_sc_body)
    return f(label_probs, table)


# speedup vs baseline: 1.0415x; 1.0187x over previous
import functools

import jax
import jax.numpy as jnp
from jax import lax
from jax.experimental import pallas as pl
from jax.experimental.pallas import tpu as pltpu
from jax.experimental.pallas import tpu_sc as plsc

N = 100000
D = 128
LANES = 16
NCORES = 2
NSUB = 16
NW = NCORES * NSUB
CHUNK = 400
NCHUNK = N // CHUNK
GROUPS = CHUNK // LANES
MAXC = NCHUNK // NW + 1
RING = 2


def _sc_body(p_hbm, tab_hbm, out_hbm, p_v, o_v, t_v, osem):
    w = lax.axis_index("s") * NCORES + lax.axis_index("c")
    nc = (NCHUNK - w + NW - 1) // NW
    s0 = jnp.where(w < 26, 8 * w, 7 * w + 26)
    start = jnp.minimum(s0, NCHUNK - MAXC)
    delta = s0 - start

    tcopy = pltpu.async_copy(tab_hbm.at[pl.ds(0, 2)], t_v, osem)
    pcopy = pltpu.async_copy(
        p_hbm.at[pl.ds(start * CHUNK, MAXC * CHUNK)], p_v, osem)
    tcopy.wait()
    pcopy.wait()

    t0s = [t_v[0, pl.ds(16 * j, 16)] for j in range(D // LANES)]
    dls = [t_v[1, pl.ds(16 * j, 16)] - t0s[j] for j in range(D // LANES)]

    H0 = 208
    H1 = CHUNK - H0

    def chunk_body(k, carry):
        off = lax.rem(k, RING) * CHUNK
        pb = p_v.at[pl.ds((k + delta) * CHUNK, CHUNK)]
        ob = o_v.at[pl.ds(off, CHUNK)]

        def group_range(g0, g1):
            def group_body(g, c):
                pv = pb[pl.ds(LANES * g, LANES)]
                for r in range(LANES):
                    pi = pv[r]
                    i = LANES * g + r
                    for j in range(D // LANES):
                        ob[i, pl.ds(16 * j, 16)] = t0s[j] + pi * dls[j]
                return c
            lax.fori_loop(g0, g1, group_body, 0)

        @pl.when(k >= RING)
        def _():
            pltpu.make_async_copy(
                ob.at[pl.ds(0, H0)], out_hbm.at[pl.ds(0, H0)], osem).wait()

        group_range(0, H0 // LANES)
        pltpu.async_copy(
            ob.at[pl.ds(0, H0)],
            out_hbm.at[pl.ds((s0 + k) * CHUNK, H0)], osem)

        @pl.when(k >= RING)
        def _():
            pltpu.make_async_copy(
                ob.at[pl.ds(H0, H1)], out_hbm.at[pl.ds(0, H1)], osem).wait()

        group_range(H0 // LANES, GROUPS)
        pltpu.async_copy(
            ob.at[pl.ds(H0, H1)],
            out_hbm.at[pl.ds((s0 + k) * CHUNK + H0, H1)], osem)
        return carry

    lax.fori_loop(0, nc, chunk_body, 0)

    ob0 = o_v.at[pl.ds(0, CHUNK)]
    for _ in range(RING):
        pltpu.make_async_copy(
            ob0.at[pl.ds(0, H0)], out_hbm.at[pl.ds(0, H0)], osem).wait()
        pltpu.make_async_copy(
            ob0.at[pl.ds(H0, H1)], out_hbm.at[pl.ds(0, H1)], osem).wait()


def kernel(label_probs, table):
    mesh = plsc.VectorSubcoreMesh(core_axis_name="c", subcore_axis_name="s")
    f = functools.partial(
        pl.kernel,
        out_type=jax.ShapeDtypeStruct((N, D), jnp.float32),
        mesh=mesh,
        scratch_types=[
            pltpu.VMEM((MAXC * CHUNK,), jnp.float32),
            pltpu.VMEM((RING * CHUNK, D), jnp.float32),
            pltpu.VMEM((2, D), jnp.float32),
            pltpu.SemaphoreType.DMA,
        ],
    )(_sc_body)
    return f(label_probs, table)
